# baseline (speedup 1.0000x reference)
import functools

import jax
import jax.numpy as jnp
from jax import lax
from jax.experimental import pallas as pl
from jax.experimental.pallas import tpu as pltpu
from jax.experimental.pallas import tpu_sc as plsc

N_EDGES = 500000
NC, NS, L = 2, 16, 16
NW = NC * NS
C = 128
CPW = 124
N_PAD = NW * CPW * C
D = 128
EPS2 = 1e-16


def _rsqrt_nr(x):
    i = plsc.bitcast(x, jnp.int32)
    y = plsc.bitcast(jnp.int32(0x5F3759DF) - (i >> 1), jnp.float32)
    for _ in range(3):
        y = y * (1.5 - 0.5 * x * y * y)
    return y


def _sc_cosine(x_user, x_job, idx_s, idx_d):
    mesh = plsc.VectorSubcoreMesh(core_axis_name="c", subcore_axis_name="s")

    @functools.partial(
        pl.kernel,
        mesh=mesh,
        compiler_params=pltpu.CompilerParams(needs_layout_passes=False),
        out_type=jax.ShapeDtypeStruct((N_PAD,), jnp.float32),
        scratch_types=[
            pltpu.VMEM((C,), jnp.int32),
            pltpu.VMEM((C,), jnp.int32),
            pltpu.VMEM((C, D), jnp.float32),
            pltpu.VMEM((C, D), jnp.float32),
            pltpu.VMEM((C,), jnp.float32),
            pltpu.SemaphoreType.DMA,
        ],
    )
    def k(xu_hbm, xj_hbm, is_hbm, id_hbm, out_hbm,
          idx_sv, idx_dv, rows_s, rows_d, out_v, sem):
        wid = lax.axis_index("s") * NC + lax.axis_index("c")
        lane = lax.iota(jnp.int32, L)

        def chunk_body(c, _):
            base = wid * (CPW * C) + c * C
            pltpu.sync_copy(is_hbm.at[pl.ds(base, C)], idx_sv)
            pltpu.sync_copy(id_hbm.at[pl.ds(base, C)], idx_dv)
            cp1 = pltpu.async_copy(xu_hbm.at[idx_sv], rows_s, sem)
            cp2 = pltpu.async_copy(xj_hbm.at[idx_dv], rows_d, sem)
            cp1.wait()
            cp2.wait()

            def eb_body(eb, _):
                row = eb * L + lane

                def k_body(ko, accs):
                    d_acc, s_acc, t_acc = accs
                    for j in range(8):
                        col = jnp.full((L,), ko * 8 + j, jnp.int32)
                        xs = plsc.load_gather(rows_s, [row, col])
                        xd = plsc.load_gather(rows_d, [row, col])
                        d_acc = d_acc + xs * xd
                        s_acc = s_acc + xs * xs
                        t_acc = t_acc + xd * xd
                    return (d_acc, s_acc, t_acc)

                z = jnp.zeros((L,), jnp.float32)
                dot, ns, nd = lax.fori_loop(0, D // 8, k_body, (z, z, z))
                rs = _rsqrt_nr(jnp.maximum(ns, EPS2))
                rd = _rsqrt_nr(jnp.maximum(nd, EPS2))
                out_v[pl.ds(eb * L, L)] = dot * rs * rd
                return _

            lax.fori_loop(0, C // L, eb_body, None)
            pltpu.sync_copy(out_v, out_hbm.at[pl.ds(base, C)])
            return _

        lax.fori_loop(0, CPW, chunk_body, None)

    return k(x_user, x_job, idx_s, idx_d)


def kernel(x_user, x_job, edge_label_index):
    idx = edge_label_index.astype(jnp.int32)
    pad = N_PAD - N_EDGES
    idx_s = jnp.pad(idx[0], (0, pad))
    idx_d = jnp.pad(idx[1], (0, pad))
    out = _sc_cosine(x_user, x_job, idx_s, idx_d)
    return out[:N_EDGES]


# speedup vs baseline: 1.3657x; 1.3657x over previous
import functools

import jax
import jax.numpy as jnp
from jax import lax
from jax.experimental import pallas as pl
from jax.experimental.pallas import tpu as pltpu
from jax.experimental.pallas import tpu_sc as plsc

N_EDGES = 500000
NC, NS, L = 2, 16, 16
NW = NC * NS
C = 128
CPW = 124
EPW = CPW * C
N_PAD = NW * EPW
D = 128
EPS2 = 1e-16


def _rsqrt_nr(x):
    i = plsc.bitcast(x, jnp.int32)
    y = plsc.bitcast(jnp.int32(0x5F3759DF) - (i >> 1), jnp.float32)
    for _ in range(3):
        y = y * (1.5 - 0.5 * x * y * y)
    return y


def _sc_cosine(x_user, x_job, idx_s, idx_d):
    mesh = plsc.VectorSubcoreMesh(core_axis_name="c", subcore_axis_name="s")

    @functools.partial(
        pl.kernel,
        mesh=mesh,
        compiler_params=pltpu.CompilerParams(needs_layout_passes=False),
        out_type=jax.ShapeDtypeStruct((N_PAD,), jnp.float32),
        scratch_types=[
            pltpu.VMEM((EPW,), jnp.int32),
            pltpu.VMEM((EPW,), jnp.int32),
            pltpu.VMEM((2, C, D), jnp.float32),
            pltpu.VMEM((2, C, D), jnp.float32),
            pltpu.VMEM((EPW,), jnp.float32),
            pltpu.SemaphoreType.DMA,
            pltpu.SemaphoreType.DMA,
            pltpu.SemaphoreType.DMA,
        ],
    )
    def k(xu_hbm, xj_hbm, is_hbm, id_hbm, out_hbm,
          idx_sv, idx_dv, rows_s, rows_d, out_v, g0, g1, si):
        wid = lax.axis_index("s") * NC + lax.axis_index("c")
        wbase = wid * EPW
        lane = lax.iota(jnp.int32, L)
        gsem = (g0, g1)

        ci1 = pltpu.async_copy(is_hbm.at[pl.ds(wbase, EPW)], idx_sv, si)
        ci2 = pltpu.async_copy(id_hbm.at[pl.ds(wbase, EPW)], idx_dv, si)
        ci1.wait()
        ci2.wait()

        def issue(c, b, sem):
            pltpu.async_copy(
                xu_hbm.at[idx_sv.at[pl.ds(c * C, C)]], rows_s.at[b], sem)
            pltpu.async_copy(
                xj_hbm.at[idx_dv.at[pl.ds(c * C, C)]], rows_d.at[b], sem)

        def wait(b, sem):
            pltpu.make_async_copy(
                xu_hbm.at[idx_sv.at[pl.ds(0, C)]], rows_s.at[b], sem).wait()
            pltpu.make_async_copy(
                xj_hbm.at[idx_dv.at[pl.ds(0, C)]], rows_d.at[b], sem).wait()

        def compute(c, b):
            rs_ref = rows_s.at[b]
            rd_ref = rows_d.at[b]

            def eb_body(eb, _):
                row = eb * L + lane

                def k_body(ko, accs):
                    d_acc, s_acc, t_acc = accs
                    for j in range(8):
                        col = jnp.full((L,), ko * 8 + j, jnp.int32)
                        xs = plsc.load_gather(rs_ref, [row, col])
                        xd = plsc.load_gather(rd_ref, [row, col])
                        d_acc = d_acc + xs * xd
                        s_acc = s_acc + xs * xs
                        t_acc = t_acc + xd * xd
                    return (d_acc, s_acc, t_acc)

                z = jnp.zeros((L,), jnp.float32)
                dot, ns, nd = lax.fori_loop(0, D // 8, k_body, (z, z, z))
                rs = _rsqrt_nr(jnp.maximum(ns, EPS2))
                rd = _rsqrt_nr(jnp.maximum(nd, EPS2))
                out_v[pl.ds(c * C + eb * L, L)] = dot * rs * rd
                return _

            lax.fori_loop(0, C // L, eb_body, None)

        issue(0, 0, g0)

        def pair_body(cc, _):
            c0 = cc * 2
            issue(c0 + 1, 1, g1)
            wait(0, g0)
            compute(c0, 0)

            @pl.when(cc < CPW // 2 - 1)
            def _prefetch():
                issue(c0 + 2, 0, g0)

            wait(1, g1)
            compute(c0 + 1, 1)
            return _

        lax.fori_loop(0, CPW // 2, pair_body, None)
        pltpu.sync_copy(out_v, out_hbm.at[pl.ds(wbase, EPW)])

    return k(x_user, x_job, idx_s, idx_d)


def kernel(x_user, x_job, edge_label_index):
    idx = edge_label_index.astype(jnp.int32)
    pad = N_PAD - N_EDGES
    idx_s = jnp.pad(idx[0], (0, pad))
    idx_d = jnp.pad(idx[1], (0, pad))
    out = _sc_cosine(x_user, x_job, idx_s, idx_d)
    return out[:N_EDGES]


# speedup vs baseline: 3.9729x; 2.9090x over previous
import functools

import jax
import jax.numpy as jnp
from jax import lax
from jax.experimental import pallas as pl
from jax.experimental.pallas import tpu as pltpu
from jax.experimental.pallas import tpu_sc as plsc

N_EDGES = 500000
NC, NS, L = 2, 16, 16
NW = NC * NS
C = 128
CPW = 124
EPW = CPW * C
N_PAD = NW * EPW
D = 128
EPS2 = 1e-16


def _rsqrt_nr(x):
    i = plsc.bitcast(x, jnp.int32)
    y = plsc.bitcast(jnp.int32(0x5F3759DF) - (i >> 1), jnp.float32)
    for _ in range(3):
        y = y * (1.5 - 0.5 * x * y * y)
    return y


def _sc_cosine(x_user, x_job, idx_s, idx_d):
    mesh = plsc.VectorSubcoreMesh(core_axis_name="c", subcore_axis_name="s")

    @functools.partial(
        pl.kernel,
        mesh=mesh,
        compiler_params=pltpu.CompilerParams(needs_layout_passes=False),
        out_type=jax.ShapeDtypeStruct((N_PAD,), jnp.float32),
        scratch_types=[
            pltpu.VMEM((EPW,), jnp.int32),
            pltpu.VMEM((EPW,), jnp.int32),
            pltpu.VMEM((2, C, D), jnp.float32),
            pltpu.VMEM((2, C, D), jnp.float32),
            pltpu.VMEM((EPW,), jnp.float32),
            pltpu.SemaphoreType.DMA,
            pltpu.SemaphoreType.DMA,
            pltpu.SemaphoreType.DMA,
        ],
    )
    def k(xu_hbm, xj_hbm, is_hbm, id_hbm, out_hbm,
          idx_sv, idx_dv, rows_s, rows_d, out_v, g0, g1, si):
        wid = lax.axis_index("s") * NC + lax.axis_index("c")
        wbase = wid * EPW
        lane = lax.iota(jnp.int32, L)
        gsem = (g0, g1)

        ci1 = pltpu.async_copy(is_hbm.at[pl.ds(wbase, EPW)], idx_sv, si)
        ci2 = pltpu.async_copy(id_hbm.at[pl.ds(wbase, EPW)], idx_dv, si)
        ci1.wait()
        ci2.wait()

        def issue(c, b, sem):
            pltpu.async_copy(
                xu_hbm.at[idx_sv.at[pl.ds(c * C, C)]], rows_s.at[b], sem)
            pltpu.async_copy(
                xj_hbm.at[idx_dv.at[pl.ds(c * C, C)]], rows_d.at[b], sem)

        def wait(b, sem):
            pltpu.make_async_copy(
                xu_hbm.at[idx_sv.at[pl.ds(0, C)]], rows_s.at[b], sem).wait()
            pltpu.make_async_copy(
                xj_hbm.at[idx_dv.at[pl.ds(0, C)]], rows_d.at[b], sem).wait()

        fifteen = jnp.full((L,), L - 1, jnp.int32)

        def compute(c, b):
            rs_ref = rows_s.at[b]
            rd_ref = rows_d.at[b]

            def group_body(g, _):
                gbase = g * L
                z = jnp.zeros((L,), jnp.float32)
                dotv, nsv, ndv = z, z, z
                for e in range(L):
                    row = gbase + e
                    acc_d, acc_s, acc_t = z, z, z
                    for j in range(D // L):
                        xs = rs_ref[row, pl.ds(j * L, L)]
                        xd = rd_ref[row, pl.ds(j * L, L)]
                        acc_d = acc_d + xs * xd
                        acc_s = acc_s + xs * xs
                        acc_t = acc_t + xd * xd
                    td = plsc.cumsum(acc_d).at[fifteen].get(
                        mode="promise_in_bounds")
                    ts = plsc.cumsum(acc_s).at[fifteen].get(
                        mode="promise_in_bounds")
                    tt = plsc.cumsum(acc_t).at[fifteen].get(
                        mode="promise_in_bounds")
                    m = lane == e
                    dotv = jnp.where(m, td, dotv)
                    nsv = jnp.where(m, ts, nsv)
                    ndv = jnp.where(m, tt, ndv)
                rs = _rsqrt_nr(jnp.maximum(nsv, EPS2))
                rd = _rsqrt_nr(jnp.maximum(ndv, EPS2))
                out_v[pl.ds(c * C + gbase, L)] = dotv * rs * rd
                return _

            lax.fori_loop(0, C // L, group_body, None)

        issue(0, 0, g0)

        def pair_body(cc, _):
            c0 = cc * 2
            issue(c0 + 1, 1, g1)
            wait(0, g0)
            compute(c0, 0)

            @pl.when(cc < CPW // 2 - 1)
            def _prefetch():
                issue(c0 + 2, 0, g0)

            wait(1, g1)
            compute(c0 + 1, 1)
            return _

        lax.fori_loop(0, CPW // 2, pair_body, None)
        pltpu.sync_copy(out_v, out_hbm.at[pl.ds(wbase, EPW)])

    return k(x_user, x_job, idx_s, idx_d)


def kernel(x_user, x_job, edge_label_index):
    idx = edge_label_index.astype(jnp.int32)
    pad = N_PAD - N_EDGES
    idx_s = jnp.pad(idx[0], (0, pad))
    idx_d = jnp.pad(idx[1], (0, pad))
    out = _sc_cosine(x_user, x_job, idx_s, idx_d)
    return out[:N_EDGES]
